# baseline (speedup 1.0000x reference)
import functools

import jax
import jax.numpy as jnp
from jax import lax
from jax.experimental import pallas as pl
from jax.experimental.pallas import tpu as pltpu
from jax.experimental.pallas import tpu_sc as plsc

EMB = 32
BATCH = 16384
NW = 32
B_PER_W = BATCH // NW
NCHUNK = 4
CHUNK = B_PER_W // NCHUNK
GROUPS = B_PER_W // 16
LANES = 16

_mesh = plsc.VectorSubcoreMesh(core_axis_name="c", subcore_axis_name="s")


def _sc_body(edge_ref, uemb_ref, iemb_ref, ub_ref, ib_ref, out_ref,
             uidx, iidx, urows, irows, ubv, ibv, outv, sem):
    wid = lax.axis_index("s") * 2 + lax.axis_index("c")
    row0 = wid * NCHUNK

    pltpu.sync_copy(edge_ref.at[0, pl.ds(row0, NCHUNK), :], uidx)
    pltpu.sync_copy(edge_ref.at[1, pl.ds(row0, NCHUNK), :], iidx)

    copies = []
    for j in range(NCHUNK):
        copies.append(pltpu.async_copy(uemb_ref.at[uidx.at[j]], urows.at[j], sem))
        copies.append(pltpu.async_copy(iemb_ref.at[iidx.at[j]], irows.at[j], sem))
        copies.append(pltpu.async_copy(ub_ref.at[uidx.at[j]], ubv.at[j], sem))
        copies.append(pltpu.async_copy(ib_ref.at[iidx.at[j]], ibv.at[j], sem))
    for c in copies:
        c.wait()

    lane_iota = lax.iota(jnp.int32, LANES)
    for j in range(NCHUNK):
        jsplat = jnp.full((LANES,), j, jnp.int32)

        def group(g, carry, j=j, jsplat=jsplat):
            e_ids = g * LANES + lane_iota
            acc = ubv[j, pl.ds(g * LANES, LANES)] + ibv[j, pl.ds(g * LANES, LANES)]
            for d in range(EMB):
                dsplat = jnp.full((LANES,), d, jnp.int32)
                uv = plsc.load_gather(urows, [jsplat, e_ids, dsplat])
                iv = plsc.load_gather(irows, [jsplat, e_ids, dsplat])
                acc = acc + uv * iv
            outv[j, pl.ds(g * LANES, LANES)] = acc
            return carry

        lax.fori_loop(0, CHUNK // LANES, group, 0)

    pltpu.sync_copy(outv, out_ref.at[pl.ds(row0, NCHUNK), :])


@functools.partial(
    pl.kernel,
    out_type=jax.ShapeDtypeStruct((BATCH // CHUNK, CHUNK), jnp.float32),
    mesh=_mesh,
    compiler_params=pltpu.CompilerParams(
        needs_layout_passes=False, use_tc_tiling_on_sc=False),
    scratch_types=[
        pltpu.VMEM((NCHUNK, CHUNK), jnp.int32),
        pltpu.VMEM((NCHUNK, CHUNK), jnp.int32),
        pltpu.VMEM((NCHUNK, CHUNK, EMB), jnp.float32),
        pltpu.VMEM((NCHUNK, CHUNK, EMB), jnp.float32),
        pltpu.VMEM((NCHUNK, CHUNK), jnp.float32),
        pltpu.VMEM((NCHUNK, CHUNK), jnp.float32),
        pltpu.VMEM((NCHUNK, CHUNK), jnp.float32),
        pltpu.SemaphoreType.DMA,
    ],
)
def _mf_sc_kernel(edge_ref, uemb_ref, iemb_ref, ub_ref, ib_ref, out_ref,
                  uidx, iidx, urows, irows, ubv, ibv, outv, sem):
    _sc_body(edge_ref, uemb_ref, iemb_ref, ub_ref, ib_ref, out_ref,
             uidx, iidx, urows, irows, ubv, ibv, outv, sem)


def kernel(edge_index, user_emb, item_emb, user_bias, item_bias):
    edge3 = edge_index.reshape(2, BATCH // CHUNK, CHUNK).astype(jnp.int32)
    out = _mf_sc_kernel(edge3, user_emb, item_emb,
                        user_bias.reshape(-1), item_bias.reshape(-1))
    return out.reshape(BATCH)
